# Initial kernel scaffold; baseline (speedup 1.0000x reference)
import jax
import jax.numpy as jnp
from jax.experimental import pallas as pl


def kernel(x, emb_table, fc_table, lin_w, lin_b):
    raise NotImplementedError("write your pallas kernel here")


# speedup vs baseline: 1.3097x; 1.3097x over previous
import functools

import jax
import jax.numpy as jnp
from jax import lax
from jax.experimental import pallas as pl
from jax.experimental.pallas import tpu as pltpu
from jax.experimental.pallas import tpu_sc as plsc

NC = 2
NS = 16
NW = NC * NS
L = 16

BATCH = 16384
N_FIELDS = 26
K = 16

import numpy as np

ROWS_PER_W = BATCH // NW
GROUP = 16
N_GROUPS = ROWS_PER_W // GROUP
CHUNK_B = 4
CHUNK_I = CHUNK_B * N_FIELDS
CHUNKS_PER_GROUP = GROUP // CHUNK_B
CHUNKS_PER_W = ROWS_PER_W // CHUNK_B
GROUP_I = GROUP * N_FIELDS

_GDN = None


def _shuffle(x, perm):
    return lax.gather(
        x, perm[:, None],
        lax.GatherDimensionNumbers(
            offset_dims=(), collapsed_slice_dims=(0,), start_index_map=(0,)),
        (1,), mode=lax.GatherScatterMode.PROMISE_IN_BOUNDS)


def _lane_sum_all(x, perms):
    for perm in perms:
        x = x + _shuffle(x, perm)
    return x


def _fm_body(emb_hbm, x2_hbm, fc_hbm, wb_hbm, out_hbm,
             idx_v, rows_v, fc_v, out_v, wb_v, sem):
    wid = lax.axis_index("s") * NC + lax.axis_index("c")
    base_chunk = wid * CHUNKS_PER_W

    pltpu.sync_copy(x2_hbm.at[pl.ds(base_chunk, CHUNKS_PER_W)], idx_v)
    pltpu.sync_copy(wb_hbm, wb_v)

    iota = lax.iota(jnp.int32, L)
    perms = [jnp.bitwise_xor(iota, step) for step in (1, 2, 4, 8)]
    wbv = wb_v[...]
    w_vec = jnp.broadcast_to(wbv[0], (L,))
    b_vec = jnp.broadcast_to(wbv[1], (L,))

    def group(g, carry):
        cps = []
        for j in range(CHUNKS_PER_GROUP):
            irow = idx_v.at[g * CHUNKS_PER_GROUP + j]
            cps.append(pltpu.async_copy(
                emb_hbm.at[irow], rows_v.at[pl.ds(j * CHUNK_I, CHUNK_I)], sem))
            cps.append(pltpu.async_copy(
                fc_hbm.at[irow], fc_v.at[pl.ds(j * CHUNK_I, CHUNK_I)], sem))
        for c in cps:
            c.wait()

        z = jnp.zeros((L,), jnp.float32)
        for r in range(GROUP):
            b0 = r * N_FIELDS
            v = rows_v[b0]
            acc = v
            accsq = v * v
            for f in range(1, N_FIELDS):
                v = rows_v[b0 + f]
                acc = acc + v
                accsq = accsq + v * v
            d = acc * acc - accsq
            f1 = fc_v[pl.ds(b0, L)]
            f2 = fc_v[pl.ds(b0 + N_FIELDS - L, L)]
            f2 = jnp.where(iota >= (2 * L - N_FIELDS), f2, 0.0)
            row_vec = 0.5 * d + w_vec * (f1 + f2)
            s = _lane_sum_all(row_vec, perms)
            z = jnp.where(iota == r, s, z)

        zb = z + b_vec
        out_v[pl.ds(g * GROUP, GROUP)] = 1.0 / (1.0 + jnp.exp(-zb))
        return carry

    lax.fori_loop(0, N_GROUPS, group, 0, unroll=False)
    pltpu.sync_copy(out_v, out_hbm.at[pl.ds(wid * ROWS_PER_W, ROWS_PER_W)])


_fm_kernel = functools.partial(
    pl.kernel,
    out_type=jax.ShapeDtypeStruct((BATCH,), jnp.float32),
    mesh=plsc.VectorSubcoreMesh(core_axis_name="c", subcore_axis_name="s"),
    compiler_params=pltpu.CompilerParams(use_tc_tiling_on_sc=False),
    scratch_types=[
        pltpu.VMEM((CHUNKS_PER_W, CHUNK_I), jnp.int32),
        pltpu.VMEM((GROUP_I, K), jnp.float32),
        pltpu.VMEM((GROUP_I,), jnp.float32),
        pltpu.VMEM((ROWS_PER_W,), jnp.float32),
        pltpu.VMEM((L,), jnp.float32),
        pltpu.SemaphoreType.DMA,
    ],
)(_fm_body)


@jax.jit
def kernel(x, emb_table, fc_table, lin_w, lin_b):
    x2 = x.astype(jnp.int32).reshape(BATCH // CHUNK_B, CHUNK_I)
    wb = jnp.zeros((L,), jnp.float32)
    wb = wb.at[0].set(lin_w[0, 0]).at[1].set(lin_b[0])
    out = _fm_kernel(emb_table, x2, fc_table.reshape(-1), wb)
    return out.reshape(BATCH, 1)


# speedup vs baseline: 1.3690x; 1.0452x over previous
import functools

import jax
import jax.numpy as jnp
from jax import lax
from jax.experimental import pallas as pl
from jax.experimental.pallas import tpu as pltpu
from jax.experimental.pallas import tpu_sc as plsc

NC = 2
NS = 16
NW = NC * NS
L = 16

BATCH = 16384
N_FIELDS = 26
K = 16

import numpy as np

ROWS_PER_W = BATCH // NW
GROUP = 16
N_GROUPS = ROWS_PER_W // GROUP
CHUNK_B = 4
CHUNK_I = CHUNK_B * N_FIELDS
CHUNKS_PER_GROUP = GROUP // CHUNK_B
CHUNKS_PER_W = ROWS_PER_W // CHUNK_B
GROUP_I = GROUP * N_FIELDS

_GDN = None


def _shuffle(x, perm):
    return lax.gather(
        x, perm[:, None],
        lax.GatherDimensionNumbers(
            offset_dims=(), collapsed_slice_dims=(0,), start_index_map=(0,)),
        (1,), mode=lax.GatherScatterMode.PROMISE_IN_BOUNDS)


def _lane_sum_all(x, perms):
    for perm in perms:
        x = x + _shuffle(x, perm)
    return x


def _fm_body(emb_hbm, x2_hbm, fc_hbm, wb_hbm, out_hbm,
             idx_v, rows_a, rows_b, fc_a, fc_b, out_v, wb_v, sem_a, sem_b):
    wid = lax.axis_index("s") * NC + lax.axis_index("c")
    base_chunk = wid * CHUNKS_PER_W

    pltpu.sync_copy(x2_hbm.at[pl.ds(base_chunk, CHUNKS_PER_W)], idx_v)
    pltpu.sync_copy(wb_hbm, wb_v)

    iota = lax.iota(jnp.int32, L)
    perms = [jnp.bitwise_xor(iota, step) for step in (1, 2, 4, 8)]
    wbv = wb_v[...]
    w_vec = jnp.broadcast_to(wbv[0], (L,))
    b_vec = jnp.broadcast_to(wbv[1], (L,))

    def issue(rows_v, fc_v, sem, g):
        for j in range(CHUNKS_PER_GROUP):
            irow = idx_v.at[g * CHUNKS_PER_GROUP + j]
            pltpu.async_copy(
                emb_hbm.at[irow], rows_v.at[pl.ds(j * CHUNK_I, CHUNK_I)], sem)
            pltpu.async_copy(
                fc_hbm.at[irow], fc_v.at[pl.ds(j * CHUNK_I, CHUNK_I)], sem)

    def drain(rows_v, fc_v, sem):
        for j in range(CHUNKS_PER_GROUP):
            pltpu.make_async_copy(
                emb_hbm.at[pl.ds(0, CHUNK_I)],
                rows_v.at[pl.ds(j * CHUNK_I, CHUNK_I)], sem).wait()
            pltpu.make_async_copy(
                fc_hbm.at[pl.ds(0, CHUNK_I)],
                fc_v.at[pl.ds(j * CHUNK_I, CHUNK_I)], sem).wait()

    def compute(rows_v, fc_v, g):
        z = jnp.zeros((L,), jnp.float32)
        for r in range(GROUP):
            b0 = r * N_FIELDS
            v = rows_v[b0]
            acc = v
            accsq = v * v
            for f in range(1, N_FIELDS):
                v = rows_v[b0 + f]
                acc = acc + v
                accsq = accsq + v * v
            d = acc * acc - accsq
            f1 = fc_v[pl.ds(b0, L)]
            f2 = fc_v[pl.ds(b0 + N_FIELDS - L, L)]
            f2 = jnp.where(iota >= (2 * L - N_FIELDS), f2, 0.0)
            row_vec = 0.5 * d + w_vec * (f1 + f2)
            s = _lane_sum_all(row_vec, perms)
            z = jnp.where(iota == r, s, z)

        zb = z + b_vec
        out_v[pl.ds(g * GROUP, GROUP)] = 1.0 / (1.0 + jnp.exp(-zb))

    issue(rows_a, fc_a, sem_a, jnp.int32(0))

    def pair(gg, carry):
        g0 = 2 * gg
        issue(rows_b, fc_b, sem_b, g0 + 1)
        drain(rows_a, fc_a, sem_a)
        compute(rows_a, fc_a, g0)
        issue(rows_a, fc_a, sem_a, lax.rem(g0 + 2, N_GROUPS))
        drain(rows_b, fc_b, sem_b)
        compute(rows_b, fc_b, g0 + 1)
        return carry

    lax.fori_loop(0, N_GROUPS // 2, pair, 0, unroll=False)
    drain(rows_a, fc_a, sem_a)
    pltpu.sync_copy(out_v, out_hbm.at[pl.ds(wid * ROWS_PER_W, ROWS_PER_W)])


_fm_kernel = functools.partial(
    pl.kernel,
    out_type=jax.ShapeDtypeStruct((BATCH,), jnp.float32),
    mesh=plsc.VectorSubcoreMesh(core_axis_name="c", subcore_axis_name="s"),
    compiler_params=pltpu.CompilerParams(use_tc_tiling_on_sc=False),
    scratch_types=[
        pltpu.VMEM((CHUNKS_PER_W, CHUNK_I), jnp.int32),
        pltpu.VMEM((GROUP_I, K), jnp.float32),
        pltpu.VMEM((GROUP_I, K), jnp.float32),
        pltpu.VMEM((GROUP_I,), jnp.float32),
        pltpu.VMEM((GROUP_I,), jnp.float32),
        pltpu.VMEM((ROWS_PER_W,), jnp.float32),
        pltpu.VMEM((L,), jnp.float32),
        pltpu.SemaphoreType.DMA,
        pltpu.SemaphoreType.DMA,
    ],
)(_fm_body)


@jax.jit
def kernel(x, emb_table, fc_table, lin_w, lin_b):
    x2 = x.astype(jnp.int32).reshape(BATCH // CHUNK_B, CHUNK_I)
    wb = jnp.zeros((L,), jnp.float32)
    wb = wb.at[0].set(lin_w[0, 0]).at[1].set(lin_b[0])
    out = _fm_kernel(emb_table, x2, fc_table.reshape(-1), wb)
    return out.reshape(BATCH, 1)
